# Initial kernel scaffold; baseline (speedup 1.0000x reference)
import functools

import jax
import jax.numpy as jnp
from jax import lax
from jax.experimental import pallas as pl
from jax.experimental.pallas import tpu as pltpu
from jax.experimental.pallas import tpu_sc as plsc

N = 10000
E = 320000
D_IN = 128
D_H = 64
D_O = 2

NC = 2
NS = 16
CHUNK = 128
N_CHUNKS = -(-E // (NC * NS * CHUNK))
EP = NC * NS * CHUNK * N_CHUNKS
EDGES_PER_TILE = CHUNK * N_CHUNKS

NROW = 10240
ROWS_PER_TILE = NROW // NS
JUNK = N
W2P = 16

_mesh = plsc.VectorSubcoreMesh(
    core_axis_name="c", subcore_axis_name="s", num_cores=NC, num_subcores=NS
)


def _zero_vmem_2d(ref, rows, width):
    def body(r, carry):
        for k in range(width // 16):
            ref[r, pl.ds(k * 16, 16)] = jnp.zeros((16,), jnp.float32)
        return carry
    lax.fori_loop(0, rows, body, 0)


@functools.partial(
    pl.kernel,
    out_type=jax.ShapeDtypeStruct((NC, NROW), jnp.float32),
    mesh=_mesh,
    scratch_types=[
        pltpu.VMEM((CHUNK,), jnp.int32),
        pltpu.VMEM((CHUNK,), jnp.float32),
        pltpu.VMEM_SHARED((NROW,), jnp.float32),
    ],
)
def _deg_kernel(dst_hbm, out_hbm, idx_v, ones_v, deg_sh):
    c = lax.axis_index("c")
    s = lax.axis_index("s")
    for k in range(CHUNK // 16):
        ones_v[pl.ds(k * 16, 16)] = jnp.zeros((16,), jnp.float32)
    for r in range(ROWS_PER_TILE // CHUNK):
        pltpu.sync_copy(ones_v, deg_sh.at[pl.ds(s * ROWS_PER_TILE + r * CHUNK, CHUNK)])
    for k in range(CHUNK // 16):
        ones_v[pl.ds(k * 16, 16)] = jnp.ones((16,), jnp.float32)
    plsc.subcore_barrier()

    tile_base = (c * NS + s) * EDGES_PER_TILE

    def body(j, carry):
        b = tile_base + j * CHUNK
        pltpu.sync_copy(dst_hbm.at[pl.ds(b, CHUNK)], idx_v)
        pltpu.sync_copy(ones_v, deg_sh.at[idx_v], add=True)
        return carry

    lax.fori_loop(0, N_CHUNKS, body, 0)
    plsc.subcore_barrier()
    pltpu.sync_copy(
        deg_sh.at[pl.ds(s * ROWS_PER_TILE, ROWS_PER_TILE)],
        out_hbm.at[c, pl.ds(s * ROWS_PER_TILE, ROWS_PER_TILE)],
    )


def _make_spmm(width):
    @functools.partial(
        pl.kernel,
        out_type=jax.ShapeDtypeStruct((NC, NROW, width), jnp.float32),
        mesh=_mesh,
        scratch_types=[
            pltpu.VMEM((CHUNK,), jnp.int32),
            pltpu.VMEM((CHUNK,), jnp.int32),
            pltpu.VMEM((CHUNK, width), jnp.float32),
            pltpu.VMEM_SHARED((NROW, width), jnp.float32),
            pltpu.SemaphoreType.DMA,
        ],
    )
    def spmm(src_hbm, dst_hbm, tab_hbm, out_hbm, idx_s, idx_d, rows_v, acc_sh, sem):
        c = lax.axis_index("c")
        s = lax.axis_index("s")
        _zero_vmem_2d(rows_v, CHUNK, width)
        for r in range(ROWS_PER_TILE // CHUNK):
            pltpu.sync_copy(
                rows_v, acc_sh.at[pl.ds(s * ROWS_PER_TILE + r * CHUNK, CHUNK)]
            )
        plsc.subcore_barrier()

        tile_base = (c * NS + s) * EDGES_PER_TILE

        def body(j, carry):
            b = tile_base + j * CHUNK
            pltpu.sync_copy(src_hbm.at[pl.ds(b, CHUNK)], idx_s)
            pltpu.sync_copy(dst_hbm.at[pl.ds(b, CHUNK)], idx_d)
            pltpu.async_copy(tab_hbm.at[idx_s], rows_v, sem).wait()
            pltpu.sync_copy(rows_v, acc_sh.at[idx_d], add=True)
            return carry

        lax.fori_loop(0, N_CHUNKS, body, 0)
        plsc.subcore_barrier()
        pltpu.sync_copy(
            acc_sh.at[pl.ds(s * ROWS_PER_TILE, ROWS_PER_TILE)],
            out_hbm.at[c, pl.ds(s * ROWS_PER_TILE, ROWS_PER_TILE)],
        )

    return spmm


_spmm_h = _make_spmm(D_H)
_spmm_o = _make_spmm(W2P)

R_BLK = 1000
GRID = N // R_BLK


def _dinv_of(degt_ref):
    deg = degt_ref[:, 0:1] + degt_ref[:, 1:2] + 1.0
    return lax.rsqrt(deg)


def _pre_body(x_ref, w1_ref, degt_ref, hs_ref):
    dinv = _dinv_of(degt_ref)
    h = jnp.dot(x_ref[...], w1_ref[...], preferred_element_type=jnp.float32)
    hs_ref[...] = h * dinv


def _mid_body(e1_ref, hs_ref, degt_ref, w2_ref, b1_ref, gsp_ref):
    dinv = _dinv_of(degt_ref)
    acc = e1_ref[0] + e1_ref[1]
    z = jnp.maximum(dinv * (acc + hs_ref[...]) + b1_ref[...], 0.0)
    g = jnp.dot(z, w2_ref[...], preferred_element_type=jnp.float32)
    gs = g * dinv
    gsp_ref[...] = jnp.concatenate(
        [gs, jnp.zeros((R_BLK, W2P - D_O), jnp.float32)], axis=1
    )


def _out_body(e2_ref, gsp_ref, degt_ref, b2_ref, o_ref):
    dinv = _dinv_of(degt_ref)
    acc = e2_ref[0] + e2_ref[1]
    o = dinv * (acc[:, 0:D_O] + gsp_ref[:, 0:D_O]) + b2_ref[...]
    m = jnp.max(o, axis=1, keepdims=True)
    lse = m + jnp.log(jnp.sum(jnp.exp(o - m), axis=1, keepdims=True))
    o_ref[...] = o - lse


@jax.jit
def kernel(x, edge_index, W1, b1, W2, b2):
    src = edge_index[0].astype(jnp.int32)
    dst = edge_index[1].astype(jnp.int32)
    pad = EP - E
    src_p = jnp.concatenate([src, jnp.zeros((pad,), jnp.int32)])
    dst_p = jnp.concatenate([dst, jnp.full((pad,), JUNK, jnp.int32)])

    deg_parts = _deg_kernel(dst_p)
    deg_t = jnp.transpose(deg_parts)

    hs = pl.pallas_call(
        _pre_body,
        grid=(GRID,),
        in_specs=[
            pl.BlockSpec((R_BLK, D_IN), lambda i: (i, 0)),
            pl.BlockSpec((D_IN, D_H), lambda i: (0, 0)),
            pl.BlockSpec((R_BLK, NC), lambda i: (i, 0)),
        ],
        out_specs=pl.BlockSpec((R_BLK, D_H), lambda i: (i, 0)),
        out_shape=jax.ShapeDtypeStruct((N, D_H), jnp.float32),
    )(x, W1, deg_t)

    eacc1 = _spmm_h(src_p, dst_p, hs)

    gsp = pl.pallas_call(
        _mid_body,
        grid=(GRID,),
        in_specs=[
            pl.BlockSpec((NC, R_BLK, D_H), lambda i: (0, i, 0)),
            pl.BlockSpec((R_BLK, D_H), lambda i: (i, 0)),
            pl.BlockSpec((R_BLK, NC), lambda i: (i, 0)),
            pl.BlockSpec((D_H, D_O), lambda i: (0, 0)),
            pl.BlockSpec((1, D_H), lambda i: (0, 0)),
        ],
        out_specs=pl.BlockSpec((R_BLK, W2P), lambda i: (i, 0)),
        out_shape=jax.ShapeDtypeStruct((N, W2P), jnp.float32),
    )(eacc1, hs, deg_t, W2, b1.reshape(1, D_H))

    eacc2 = _spmm_o(src_p, dst_p, gsp)

    out = pl.pallas_call(
        _out_body,
        grid=(GRID,),
        in_specs=[
            pl.BlockSpec((NC, R_BLK, W2P), lambda i: (0, i, 0)),
            pl.BlockSpec((R_BLK, W2P), lambda i: (i, 0)),
            pl.BlockSpec((R_BLK, NC), lambda i: (i, 0)),
            pl.BlockSpec((1, D_O), lambda i: (0, 0)),
        ],
        out_specs=pl.BlockSpec((R_BLK, D_O), lambda i: (i, 0)),
        out_shape=jax.ShapeDtypeStruct((N, D_O), jnp.float32),
    )(eacc2, gsp, deg_t, b2.reshape(1, D_O))

    return out


# speedup vs baseline: 17.0918x; 17.0918x over previous
import functools

import jax
import jax.numpy as jnp
from jax import lax
from jax.experimental import pallas as pl
from jax.experimental.pallas import tpu as pltpu
from jax.experimental.pallas import tpu_sc as plsc

N = 10000
E = 320000
D_IN = 128
D_H = 64
D_O = 2

NC = 2
NS = 16
CHUNK = 128
N_CHUNKS = -(-E // (NC * NS * CHUNK))
EP = NC * NS * CHUNK * N_CHUNKS
EDGES_PER_TILE = CHUNK * N_CHUNKS

NROW = 10240
ROWS_PER_TILE = NROW // NS
JUNK = N
W2P = 16

_mesh = plsc.VectorSubcoreMesh(
    core_axis_name="c", subcore_axis_name="s", num_cores=NC, num_subcores=NS
)


def _zero_vmem_2d(ref, rows, width):
    def body(r, carry):
        for k in range(width // 16):
            ref[r, pl.ds(k * 16, 16)] = jnp.zeros((16,), jnp.float32)
        return carry
    lax.fori_loop(0, rows, body, 0)


@functools.partial(
    pl.kernel,
    out_type=jax.ShapeDtypeStruct((NC, NROW), jnp.float32),
    mesh=_mesh,
    compiler_params=pltpu.CompilerParams(use_tc_tiling_on_sc=False),
    scratch_types=[
        pltpu.VMEM((CHUNK,), jnp.int32),
        pltpu.VMEM((CHUNK,), jnp.float32),
        pltpu.VMEM_SHARED((NROW,), jnp.float32),
    ],
)
def _deg_kernel(dst_hbm, out_hbm, idx_v, ones_v, deg_sh):
    c = lax.axis_index("c")
    s = lax.axis_index("s")
    for k in range(CHUNK // 16):
        ones_v[pl.ds(k * 16, 16)] = jnp.zeros((16,), jnp.float32)
    for r in range(ROWS_PER_TILE // CHUNK):
        pltpu.sync_copy(ones_v, deg_sh.at[pl.ds(s * ROWS_PER_TILE + r * CHUNK, CHUNK)])
    for k in range(CHUNK // 16):
        ones_v[pl.ds(k * 16, 16)] = jnp.ones((16,), jnp.float32)
    plsc.subcore_barrier()

    tile_base = (c * NS + s) * EDGES_PER_TILE

    def body(j, carry):
        b = tile_base + j * CHUNK
        pltpu.sync_copy(dst_hbm.at[pl.ds(b, CHUNK)], idx_v)
        pltpu.sync_copy(ones_v, deg_sh.at[idx_v], add=True)
        return carry

    lax.fori_loop(0, N_CHUNKS, body, 0)
    plsc.subcore_barrier()
    pltpu.sync_copy(
        deg_sh.at[pl.ds(s * ROWS_PER_TILE, ROWS_PER_TILE)],
        out_hbm.at[c, pl.ds(s * ROWS_PER_TILE, ROWS_PER_TILE)],
    )


def _make_spmm(width):
    @functools.partial(
        pl.kernel,
        out_type=jax.ShapeDtypeStruct((NC, NROW, width), jnp.float32),
        mesh=_mesh,
        compiler_params=pltpu.CompilerParams(use_tc_tiling_on_sc=False),
        scratch_types=[
            pltpu.VMEM((CHUNK,), jnp.int32),
            pltpu.VMEM((CHUNK,), jnp.int32),
            pltpu.VMEM((CHUNK, width), jnp.float32),
            pltpu.VMEM_SHARED((NROW, width), jnp.float32),
            pltpu.SemaphoreType.DMA,
        ],
    )
    def spmm(src_hbm, dst_hbm, tab_hbm, out_hbm, idx_s, idx_d, rows_v, acc_sh, sem):
        c = lax.axis_index("c")
        s = lax.axis_index("s")
        _zero_vmem_2d(rows_v, CHUNK, width)
        for r in range(ROWS_PER_TILE // CHUNK):
            pltpu.sync_copy(
                rows_v, acc_sh.at[pl.ds(s * ROWS_PER_TILE + r * CHUNK, CHUNK)]
            )
        plsc.subcore_barrier()

        tile_base = (c * NS + s) * EDGES_PER_TILE

        def body(j, carry):
            b = tile_base + j * CHUNK
            pltpu.sync_copy(src_hbm.at[pl.ds(b, CHUNK)], idx_s)
            pltpu.sync_copy(dst_hbm.at[pl.ds(b, CHUNK)], idx_d)
            pltpu.async_copy(tab_hbm.at[idx_s], rows_v, sem).wait()
            pltpu.sync_copy(rows_v, acc_sh.at[idx_d], add=True)
            return carry

        lax.fori_loop(0, N_CHUNKS, body, 0)
        plsc.subcore_barrier()
        pltpu.sync_copy(
            acc_sh.at[pl.ds(s * ROWS_PER_TILE, ROWS_PER_TILE)],
            out_hbm.at[c, pl.ds(s * ROWS_PER_TILE, ROWS_PER_TILE)],
        )

    return spmm


_spmm_h = _make_spmm(D_H)
_spmm_o = _make_spmm(W2P)

R_BLK = 1000
GRID = N // R_BLK


def _dinv_of(degt_ref):
    deg = degt_ref[:, 0:1] + degt_ref[:, 1:2] + 1.0
    return lax.rsqrt(deg)


def _pre_body(x_ref, w1_ref, degt_ref, hs_ref):
    dinv = _dinv_of(degt_ref)
    h = jnp.dot(x_ref[...], w1_ref[...], preferred_element_type=jnp.float32)
    hs_ref[...] = h * dinv


def _mid_body(e1_ref, hs_ref, degt_ref, w2_ref, b1_ref, gsp_ref):
    dinv = _dinv_of(degt_ref)
    acc = e1_ref[0] + e1_ref[1]
    z = jnp.maximum(dinv * (acc + hs_ref[...]) + b1_ref[...], 0.0)
    g = jnp.dot(z, w2_ref[...], preferred_element_type=jnp.float32)
    gs = g * dinv
    gsp_ref[...] = jnp.concatenate(
        [gs, jnp.zeros((R_BLK, W2P - D_O), jnp.float32)], axis=1
    )


def _out_body(e2_ref, gsp_ref, degt_ref, b2_ref, o_ref):
    dinv = _dinv_of(degt_ref)
    acc = e2_ref[0] + e2_ref[1]
    o = dinv * (acc[:, 0:D_O] + gsp_ref[:, 0:D_O]) + b2_ref[...]
    m = jnp.max(o, axis=1, keepdims=True)
    lse = m + jnp.log(jnp.sum(jnp.exp(o - m), axis=1, keepdims=True))
    o_ref[...] = o - lse


@jax.jit
def kernel(x, edge_index, W1, b1, W2, b2):
    src = edge_index[0].astype(jnp.int32)
    dst = edge_index[1].astype(jnp.int32)
    pad = EP - E
    src_p = jnp.concatenate([src, jnp.zeros((pad,), jnp.int32)])
    dst_p = jnp.concatenate([dst, jnp.full((pad,), JUNK, jnp.int32)])

    deg_parts = _deg_kernel(dst_p)
    deg_t = jnp.transpose(deg_parts)

    hs = pl.pallas_call(
        _pre_body,
        grid=(GRID,),
        in_specs=[
            pl.BlockSpec((R_BLK, D_IN), lambda i: (i, 0)),
            pl.BlockSpec((D_IN, D_H), lambda i: (0, 0)),
            pl.BlockSpec((R_BLK, NC), lambda i: (i, 0)),
        ],
        out_specs=pl.BlockSpec((R_BLK, D_H), lambda i: (i, 0)),
        out_shape=jax.ShapeDtypeStruct((N, D_H), jnp.float32),
    )(x, W1, deg_t)

    eacc1 = _spmm_h(src_p, dst_p, hs)

    gsp = pl.pallas_call(
        _mid_body,
        grid=(GRID,),
        in_specs=[
            pl.BlockSpec((NC, R_BLK, D_H), lambda i: (0, i, 0)),
            pl.BlockSpec((R_BLK, D_H), lambda i: (i, 0)),
            pl.BlockSpec((R_BLK, NC), lambda i: (i, 0)),
            pl.BlockSpec((D_H, D_O), lambda i: (0, 0)),
            pl.BlockSpec((1, D_H), lambda i: (0, 0)),
        ],
        out_specs=pl.BlockSpec((R_BLK, W2P), lambda i: (i, 0)),
        out_shape=jax.ShapeDtypeStruct((N, W2P), jnp.float32),
    )(eacc1, hs, deg_t, W2, b1.reshape(1, D_H))

    eacc2 = _spmm_o(src_p, dst_p, gsp)

    out = pl.pallas_call(
        _out_body,
        grid=(GRID,),
        in_specs=[
            pl.BlockSpec((NC, R_BLK, W2P), lambda i: (0, i, 0)),
            pl.BlockSpec((R_BLK, W2P), lambda i: (i, 0)),
            pl.BlockSpec((R_BLK, NC), lambda i: (i, 0)),
            pl.BlockSpec((1, D_O), lambda i: (0, 0)),
        ],
        out_specs=pl.BlockSpec((R_BLK, D_O), lambda i: (i, 0)),
        out_shape=jax.ShapeDtypeStruct((N, D_O), jnp.float32),
    )(eacc2, gsp, deg_t, b2.reshape(1, D_O))

    return out
